# baseline (speedup 1.0000x reference)
import jax
import jax.numpy as jnp
from jax.experimental import pallas as pl

_BR = 512
_N_CLASSES = 100
_EPS = 1e-8


def _retrieval_kernel(hvs_ref, am_ref, out_ref):
    am = am_ref[...]
    am_n = am / jnp.maximum(
        jnp.sqrt(jnp.sum(am * am, axis=1, keepdims=True)), _EPS)
    am_b = am_n.astype(jnp.bfloat16)

    x = hvs_ref[...]
    x_n = x / jnp.maximum(
        jnp.sqrt(jnp.sum(x * x, axis=1, keepdims=True)), _EPS)
    scores = jax.lax.dot_general(
        x_n.astype(jnp.bfloat16), am_b,
        dimension_numbers=(((1,), (1,)), ((), ())),
        preferred_element_type=jnp.float32,
    )

    m = jnp.max(scores, axis=1, keepdims=True)
    idx = jax.lax.broadcasted_iota(jnp.int32, scores.shape, 1)
    preds = jnp.min(jnp.where(scores == m, idx, _N_CLASSES), axis=1,
                    keepdims=True)
    out_ref[...] = preds


@jax.jit
def kernel(hvs, am):
    n_rows, d = hvs.shape
    out = pl.pallas_call(
        _retrieval_kernel,
        grid=(n_rows // _BR,),
        in_specs=[
            pl.BlockSpec((_BR, d), lambda i: (i, 0)),
            pl.BlockSpec(am.shape, lambda i: (0, 0)),
        ],
        out_specs=pl.BlockSpec((_BR, 1), lambda i: (i, 0)),
        out_shape=jax.ShapeDtypeStruct((n_rows, 1), jnp.int32),
    )(hvs, am.astype(jnp.float32))
    return out.reshape(n_rows)
